# Initial kernel scaffold; baseline (speedup 1.0000x reference)
import jax
import jax.numpy as jnp
from jax.experimental import pallas as pl


def kernel(edge_list, ev_in, ee_in, pu_in, params):
    raise NotImplementedError("write your pallas kernel here")


# speedup vs baseline: 10.3309x; 10.3309x over previous
import functools

import jax
import jax.numpy as jnp
from jax.experimental import pallas as pl

_N = 256
_D = 128
_BS = 16

_bf = jnp.bfloat16
_f32 = jnp.float32


def _mm(a_bf, w_ref):
    return jnp.dot(a_bf, w_ref[...].astype(_bf), preferred_element_type=_f32)


def _fold_dst(ep):
    ep_r = ep.reshape(_BS, _N, _D)
    acc = ep_r[:, 0, :]
    for j in range(1, _N):
        acc = acc + ep_r[:, j, :]
    return acc


def _edge_body(emit_ep, first, e_ref, vs_ref, v_ref, up_ref,
               w1a_ref, w1b_ref, b1_ref, w2_ref, b2_ref, w3_ref, b3_ref,
               w4_ref, b4_ref, *rest):
    if first:
        wee_ref, bee_ref = rest[0], rest[1]
        out_refs = rest[2:]
        e_bf = (_mm(e_ref[...].astype(_bf), wee_ref) + bee_ref[...]).astype(_bf)
    else:
        out_refs = rest
        e_bf = e_ref[...]
    vs = vs_ref[...].astype(_bf)
    vd = v_ref[...].astype(_bf)
    ub = up_ref[...].astype(_bf)
    vs_t = jnp.broadcast_to(vs[:, None, :], (_BS, _N, _D)).reshape(_BS * _N, _D)
    vd_t = jnp.broadcast_to(vd[None, :, :], (_BS, _N, _D)).reshape(_BS * _N, _D)
    u_t = jnp.broadcast_to(ub, (_BS * _N, 128))
    x1 = jnp.concatenate([e_bf, vs_t], axis=1)
    x2 = jnp.concatenate([vd_t, u_t], axis=1)
    h = _mm(x1, w1a_ref) + _mm(x2, w1b_ref)
    h = jnp.maximum(h + b1_ref[...], 0.0)
    h = jnp.maximum(_mm(h.astype(_bf), w2_ref) + b2_ref[...], 0.0)
    h = jnp.maximum(_mm(h.astype(_bf), w3_ref) + b3_ref[...], 0.0)
    ep = _mm(h.astype(_bf), w4_ref) + b4_ref[...]
    if emit_ep:
        out_refs[0][...] = ep.astype(_bf)
        out_refs[1][...] = _fold_dst(ep)
    else:
        out_refs[0][...] = _fold_dst(ep)


def _edge_round(e, v, up, w1a, w1b, b1, w2, b2, w3, b3, w4, b4,
                emit_ep, embed=None):
    de = e.shape[1]
    first = embed is not None
    args = [e, v, v, up, w1a, w1b, b1, w2, b2, w3, b3, w4, b4]
    in_specs = [
        pl.BlockSpec((_BS * _N, de), lambda i: (i, 0)),
        pl.BlockSpec((_BS, _D), lambda i: (i, 0)),
        pl.BlockSpec((_N, _D), lambda i: (0, 0)),
        pl.BlockSpec((1, 128), lambda i: (0, 0)),
    ] + [pl.BlockSpec(w.shape, lambda i: (0, 0))
         for w in (w1a, w1b, b1, w2, b2, w3, b3, w4, b4)]
    if first:
        args += list(embed)
        in_specs += [pl.BlockSpec(w.shape, lambda i: (0, 0)) for w in embed]
    if emit_ep:
        out_shape = (
            jax.ShapeDtypeStruct((_N * _N, _D), _bf),
            jax.ShapeDtypeStruct((_N, _D), _f32),
        )
        out_specs = (
            pl.BlockSpec((_BS * _N, _D), lambda i: (i, 0)),
            pl.BlockSpec((_BS, _D), lambda i: (i, 0)),
        )
    else:
        out_shape = (jax.ShapeDtypeStruct((_N, _D), _f32),)
        out_specs = (pl.BlockSpec((_BS, _D), lambda i: (i, 0)),)
    return pl.pallas_call(
        functools.partial(_edge_body, emit_ep, first),
        grid=(_N // _BS,),
        in_specs=in_specs,
        out_specs=out_specs,
        out_shape=out_shape,
    )(*args)


def _node_body(epc_ref, v_ref, up_ref,
               wv1a_ref, wv1b_ref, bv1_ref, wv2_ref, bv2_ref, wv3_ref,
               bv3_ref, wv4_ref, bv4_ref,
               wq1a_ref, wq1b_ref, bq1_ref, wq2_ref, bq2_ref, wq3_ref,
               bq3_ref, wq4_ref, bq4_ref,
               vp_ref, uo_ref):
    epc = epc_ref[...]
    ub = up_ref[...].astype(_bf)
    u_t = jnp.broadcast_to(ub, (_N, 128))
    x = jnp.concatenate([epc.astype(_bf), v_ref[...].astype(_bf)], axis=1)
    vp = _mm(x, wv1a_ref) + _mm(u_t, wv1b_ref)
    vp = jnp.maximum(vp + bv1_ref[...], 0.0)
    vp = jnp.maximum(_mm(vp.astype(_bf), wv2_ref) + bv2_ref[...], 0.0)
    vp = jnp.maximum(_mm(vp.astype(_bf), wv3_ref) + bv3_ref[...], 0.0)
    vp = _mm(vp.astype(_bf), wv4_ref) + bv4_ref[...]
    vp_ref[...] = vp
    e_bar = jnp.sum(epc, axis=0, keepdims=True)
    v_bar = jnp.sum(vp, axis=0, keepdims=True)
    x2 = jnp.concatenate([e_bar.astype(_bf), v_bar.astype(_bf)], axis=1)
    uu = _mm(x2, wq1a_ref) + _mm(ub, wq1b_ref)
    uu = jnp.maximum(uu + bq1_ref[...], 0.0)
    uu = jnp.maximum(_mm(uu.astype(_bf), wq2_ref) + bq2_ref[...], 0.0)
    uu = jnp.maximum(_mm(uu.astype(_bf), wq3_ref) + bq3_ref[...], 0.0)
    uo_ref[...] = _mm(uu.astype(_bf), wq4_ref) + bq4_ref[...]


def _node_round(epc, v, up, pv, pu):
    return pl.pallas_call(
        _node_body,
        out_shape=(
            jax.ShapeDtypeStruct((_N, _D), _f32),
            jax.ShapeDtypeStruct((1, 16), _f32),
        ),
    )(epc, v, up, *pv, *pu)


def _prologue_body(x_ref, w_ref, b_ref, v_ref):
    v_ref[...] = _mm(x_ref[...].astype(_bf), w_ref) + b_ref[...]


def _decoder_body(vp_ref, w1_ref, b1_ref, w2_ref, b2_ref, w3_ref, b3_ref,
                  w4_ref, b4_ref, y_ref):
    y = jnp.maximum(_mm(vp_ref[...].astype(_bf), w1_ref) + b1_ref[...], 0.0)
    y = jnp.maximum(_mm(y.astype(_bf), w2_ref) + b2_ref[...], 0.0)
    y = jnp.maximum(_mm(y.astype(_bf), w3_ref) + b3_ref[...], 0.0)
    y_ref[...] = _mm(y.astype(_bf), w4_ref) + b4_ref[...]


def _single(body, args, out_shape):
    return pl.pallas_call(body, out_shape=out_shape)(*args)


def kernel(edge_list, ev_in, ee_in, pu_in, params):
    p = params

    def row(name):
        return p[name + "_b"][None, :].astype(_f32)

    def padk(w, k):
        return jnp.pad(w, ((0, k - w.shape[0]), (0, 0)))

    def padu(u):
        return jnp.pad(u, ((0, 0), (0, 112)))

    w1a = p["pe1_W"][:256]
    w1b = padk(p["pe1_W"][256:], 256)
    pv = (p["pv1_W"][:256], padk(p["pv1_W"][256:], 128), row("pv1"),
          p["pv2_W"], row("pv2"), p["pv3_W"], row("pv3"),
          p["pv4_W"], row("pv4"))
    pu = (p["pu1_W"][:256], padk(p["pu1_W"][256:], 128), row("pu1"),
          p["pu2_W"], row("pu2"), p["pu3_W"], row("pu3"),
          p["pu4_W"], row("pu4"))
    common = (row("pe1"), p["pe2_W"], row("pe2"), p["pe3_W"], row("pe3"),
              p["pe4_W"], row("pe4"))

    xev = jnp.concatenate([ev_in, p["attribs"]], axis=1)
    v = _single(_prologue_body, (xev, p["ev_W"], row("ev")),
                jax.ShapeDtypeStruct((_N, _D), _f32))

    up = padu(p["u0"][None, :])

    ep, epc = _edge_round(ee_in, v, up, w1a, w1b, *common,
                          emit_ep=True, embed=(p["ee_W"], row("ee")))
    v, u = _node_round(epc, v, up, pv, pu)
    up = padu(u)
    ep, epc = _edge_round(ep, v, up, w1a, w1b, *common, emit_ep=True)
    v, u = _node_round(epc, v, up, pv, pu)
    up = padu(u)
    (epc,) = _edge_round(ep, v, up, w1a, w1b, *common, emit_ep=False)
    v, u = _node_round(epc, v, up, pv, pu)

    y = _single(_decoder_body,
                (v, p["d1_W"], row("d1"), p["d2_W"], row("d2"),
                 p["d3_W"], row("d3"), p["d4_W"], row("d4")),
                jax.ShapeDtypeStruct((_N, 3), _f32))
    return y


# speedup vs baseline: 12.7829x; 1.2374x over previous
import functools

import jax
import jax.numpy as jnp
from jax.experimental import pallas as pl

_N = 256
_D = 128
_BS = 16

_bf = jnp.bfloat16
_f32 = jnp.float32


def _mm(a_bf, w_ref):
    return jnp.dot(a_bf, w_ref[...].astype(_bf), preferred_element_type=_f32)


def _edge_body(emit_ep, first, e_ref, vd_ref, v_ref, up_ref,
               w1a_ref, w1b_ref, b1_ref, w2_ref, b2_ref, w3_ref, b3_ref,
               w4_ref, b4_ref, *rest):
    if first:
        wee_ref, bee_ref = rest[0], rest[1]
        out_refs = rest[2:]
        e_bf = (_mm(e_ref[...].astype(_bf), wee_ref) + bee_ref[...]).astype(_bf)
    else:
        out_refs = rest
        e_bf = e_ref[...]
    i = pl.program_id(0)
    vd = vd_ref[...].astype(_bf)
    vfull = v_ref[...].astype(_bf)
    ub = up_ref[...].astype(_bf)
    vs_t = jnp.broadcast_to(vfull[None, :, :], (_BS, _N, _D)).reshape(_BS * _N, _D)
    vd_t = jnp.broadcast_to(vd[:, None, :], (_BS, _N, _D)).reshape(_BS * _N, _D)
    u_t = jnp.broadcast_to(ub, (_BS * _N, 128))
    x1 = jnp.concatenate([e_bf, vs_t], axis=1)
    x2 = jnp.concatenate([vd_t, u_t], axis=1)
    h = _mm(x1, w1a_ref) + _mm(x2, w1b_ref)
    h = jnp.maximum(h + b1_ref[...], 0.0)
    h = jnp.maximum(_mm(h.astype(_bf), w2_ref) + b2_ref[...], 0.0)
    h = jnp.maximum(_mm(h.astype(_bf), w3_ref) + b3_ref[...], 0.0)
    ep = _mm(h.astype(_bf), w4_ref) + b4_ref[...]
    epc_ref = out_refs[-1]

    @pl.when(i == 0)
    def _():
        epc_ref[...] = jnp.zeros((_N, _D), _f32)

    if emit_ep:
        out_refs[0][...] = ep.astype(_bf)
    ep_r = ep.reshape(_BS, _N, _D)
    acc = epc_ref[...]
    for dd in range(_BS):
        acc = acc + ep_r[dd]
    epc_ref[...] = acc


def _edge_round(e, v, up, w1a, w1b, b1, w2, b2, w3, b3, w4, b4,
                emit_ep, embed=None):
    de = e.shape[1]
    first = embed is not None
    args = [e, v, v, up, w1a, w1b, b1, w2, b2, w3, b3, w4, b4]
    in_specs = [
        pl.BlockSpec((_BS * _N, de), lambda i: (i, 0)),
        pl.BlockSpec((_BS, _D), lambda i: (i, 0)),
        pl.BlockSpec((_N, _D), lambda i: (0, 0)),
        pl.BlockSpec((1, 128), lambda i: (0, 0)),
    ] + [pl.BlockSpec(w.shape, lambda i: (0, 0))
         for w in (w1a, w1b, b1, w2, b2, w3, b3, w4, b4)]
    if first:
        args += list(embed)
        in_specs += [pl.BlockSpec(w.shape, lambda i: (0, 0)) for w in embed]
    if emit_ep:
        out_shape = (
            jax.ShapeDtypeStruct((_N * _N, _D), _bf),
            jax.ShapeDtypeStruct((_N, _D), _f32),
        )
        out_specs = (
            pl.BlockSpec((_BS * _N, _D), lambda i: (i, 0)),
            pl.BlockSpec((_N, _D), lambda i: (0, 0)),
        )
    else:
        out_shape = (jax.ShapeDtypeStruct((_N, _D), _f32),)
        out_specs = (pl.BlockSpec((_N, _D), lambda i: (0, 0)),)
    return pl.pallas_call(
        functools.partial(_edge_body, emit_ep, first),
        grid=(_N // _BS,),
        in_specs=in_specs,
        out_specs=out_specs,
        out_shape=out_shape,
    )(*args)


def _node_body(epc_ref, v_ref, up_ref,
               wv1a_ref, wv1b_ref, bv1_ref, wv2_ref, bv2_ref, wv3_ref,
               bv3_ref, wv4_ref, bv4_ref,
               wq1a_ref, wq1b_ref, bq1_ref, wq2_ref, bq2_ref, wq3_ref,
               bq3_ref, wq4_ref, bq4_ref,
               vp_ref, uo_ref):
    epc = epc_ref[...]
    ub = up_ref[...].astype(_bf)
    u_t = jnp.broadcast_to(ub, (_N, 128))
    x = jnp.concatenate([epc.astype(_bf), v_ref[...].astype(_bf)], axis=1)
    vp = _mm(x, wv1a_ref) + _mm(u_t, wv1b_ref)
    vp = jnp.maximum(vp + bv1_ref[...], 0.0)
    vp = jnp.maximum(_mm(vp.astype(_bf), wv2_ref) + bv2_ref[...], 0.0)
    vp = jnp.maximum(_mm(vp.astype(_bf), wv3_ref) + bv3_ref[...], 0.0)
    vp = _mm(vp.astype(_bf), wv4_ref) + bv4_ref[...]
    vp_ref[...] = vp
    e_bar = jnp.sum(epc, axis=0, keepdims=True)
    v_bar = jnp.sum(vp, axis=0, keepdims=True)
    x2 = jnp.concatenate([e_bar.astype(_bf), v_bar.astype(_bf)], axis=1)
    uu = _mm(x2, wq1a_ref) + _mm(ub, wq1b_ref)
    uu = jnp.maximum(uu + bq1_ref[...], 0.0)
    uu = jnp.maximum(_mm(uu.astype(_bf), wq2_ref) + bq2_ref[...], 0.0)
    uu = jnp.maximum(_mm(uu.astype(_bf), wq3_ref) + bq3_ref[...], 0.0)
    uo_ref[...] = _mm(uu.astype(_bf), wq4_ref) + bq4_ref[...]


def _node_round(epc, v, up, pv, pu):
    return pl.pallas_call(
        _node_body,
        out_shape=(
            jax.ShapeDtypeStruct((_N, _D), _f32),
            jax.ShapeDtypeStruct((1, 16), _f32),
        ),
    )(epc, v, up, *pv, *pu)


def _prologue_body(x_ref, w_ref, b_ref, v_ref):
    v_ref[...] = _mm(x_ref[...].astype(_bf), w_ref) + b_ref[...]


def _decoder_body(vp_ref, w1_ref, b1_ref, w2_ref, b2_ref, w3_ref, b3_ref,
                  w4_ref, b4_ref, y_ref):
    y = jnp.maximum(_mm(vp_ref[...].astype(_bf), w1_ref) + b1_ref[...], 0.0)
    y = jnp.maximum(_mm(y.astype(_bf), w2_ref) + b2_ref[...], 0.0)
    y = jnp.maximum(_mm(y.astype(_bf), w3_ref) + b3_ref[...], 0.0)
    y_ref[...] = _mm(y.astype(_bf), w4_ref) + b4_ref[...]


def _single(body, args, out_shape):
    return pl.pallas_call(body, out_shape=out_shape)(*args)


def kernel(edge_list, ev_in, ee_in, pu_in, params):
    p = params

    def row(name):
        return p[name + "_b"][None, :].astype(_f32)

    def padk(w, k):
        return jnp.pad(w, ((0, k - w.shape[0]), (0, 0)))

    def padu(u):
        return jnp.pad(u, ((0, 0), (0, 112)))

    w1a = p["pe1_W"][:256]
    w1b = padk(p["pe1_W"][256:], 256)
    pv = (p["pv1_W"][:256], padk(p["pv1_W"][256:], 128), row("pv1"),
          p["pv2_W"], row("pv2"), p["pv3_W"], row("pv3"),
          p["pv4_W"], row("pv4"))
    pu = (p["pu1_W"][:256], padk(p["pu1_W"][256:], 128), row("pu1"),
          p["pu2_W"], row("pu2"), p["pu3_W"], row("pu3"),
          p["pu4_W"], row("pu4"))
    common = (row("pe1"), p["pe2_W"], row("pe2"), p["pe3_W"], row("pe3"),
              p["pe4_W"], row("pe4"))

    xev = jnp.concatenate([ev_in, p["attribs"]], axis=1)
    v = _single(_prologue_body, (xev, p["ev_W"], row("ev")),
                jax.ShapeDtypeStruct((_N, _D), _f32))

    up = padu(p["u0"][None, :])

    ee_t = ee_in.reshape(_N, _N, 5).transpose(1, 0, 2).reshape(_N * _N, 5)
    ep, epc = _edge_round(ee_t, v, up, w1a, w1b, *common,
                          emit_ep=True, embed=(p["ee_W"], row("ee")))
    v, u = _node_round(epc, v, up, pv, pu)
    up = padu(u)
    ep, epc = _edge_round(ep, v, up, w1a, w1b, *common, emit_ep=True)
    v, u = _node_round(epc, v, up, pv, pu)
    up = padu(u)
    (epc,) = _edge_round(ep, v, up, w1a, w1b, *common, emit_ep=False)
    v, u = _node_round(epc, v, up, pv, pu)

    y = _single(_decoder_body,
                (v, p["d1_W"], row("d1"), p["d2_W"], row("d2"),
                 p["d3_W"], row("d3"), p["d4_W"], row("d4")),
                jax.ShapeDtypeStruct((_N, 3), _f32))
    return y
